# baseline (speedup 1.0000x reference)
import functools

import jax
import jax.numpy as jnp
from jax import lax
from jax.experimental import pallas as pl
from jax.experimental.pallas import tpu as pltpu
from jax.experimental.pallas import tpu_sc as plsc

N_NODES = 10000
N_EDGES = 320000
D = 128

NC = 2
NS = 16
N_WORKERS = NC * NS

CHUNK = 80
EDGES_PER_WORKER = N_EDGES // N_WORKERS
GROUP = 21
NGROUPS = 6
CHUNKS_PER_WORKER = GROUP * NGROUPS
EPW_PAD = CHUNKS_PER_WORKER * CHUNK
ROWS_PER_TILE = 632
N_PAD = NS * ROWS_PER_TILE

_mesh = plsc.VectorSubcoreMesh(core_axis_name="c", subcore_axis_name="s")


@functools.partial(
    pl.kernel,
    mesh=_mesh,
    out_type=jax.ShapeDtypeStruct((NC, N_PAD, D), jnp.float32),
    scratch_types=[
        pltpu.VMEM_SHARED((N_PAD, D), jnp.float32),
        pltpu.VMEM((EPW_PAD,), jnp.int32),
        pltpu.VMEM((GROUP, CHUNK), jnp.int32),
        pltpu.VMEM((GROUP, CHUNK), jnp.int32),
        pltpu.VMEM((CHUNK, D), jnp.float32),
        pltpu.VMEM((CHUNK, D), jnp.float32),
        pltpu.VMEM((CHUNK, D), jnp.float32),
        pltpu.SemaphoreType.DMA,
        pltpu.SemaphoreType.DMA,
        pltpu.SemaphoreType.DMA,
        pltpu.SemaphoreType.DMA,
        pltpu.SemaphoreType.DMA,
        pltpu.SemaphoreType.DMA,
        pltpu.SemaphoreType.DMA,
    ],
)
def _sc_segment_sum(x_hbm, src_hbm, dst_hbm, zeros_hbm, out_hbm,
                    acc, src_v, dv0, dv1, r0, r1, r2,
                    g0, g1, g2, s0, s1, s2, psem):
    c = lax.axis_index("c")
    s = lax.axis_index("s")
    w = c * NS + s
    rows = (r0, r1, r2)
    gs = (g0, g1, g2)
    ss = (s0, s1, s2)
    dv = (dv0, dv1)

    zslice = pl.ds(s * ROWS_PER_TILE, ROWS_PER_TILE)
    eslice = pl.ds(w * EPW_PAD, EPW_PAD)
    pltpu.async_copy(zeros_hbm.at[zslice], acc.at[zslice], psem)
    pltpu.async_copy(src_hbm.at[eslice], src_v, g0)
    pltpu.async_copy(dst_hbm.at[w, 0], dv0, g1)
    pltpu.make_async_copy(zeros_hbm.at[zslice], acc.at[zslice], psem).wait()
    pltpu.make_async_copy(src_hbm.at[eslice], src_v, g0).wait()
    pltpu.make_async_copy(dst_hbm.at[w, 0], dv0, g1).wait()
    plsc.subcore_barrier()

    def _sidx(j):
        return src_v.at[pl.ds(pl.multiple_of(j * CHUNK, 8), CHUNK)]

    def gather(j, r):
        pltpu.async_copy(x_hbm.at[_sidx(j)], rows[r], gs[r])

    def gather_wait(j, r):
        pltpu.make_async_copy(x_hbm.at[_sidx(j)], rows[r], gs[r]).wait()

    def scat(r, dbuf, k):
        pltpu.async_copy(rows[r], acc.at[dbuf.at[k]], ss[r], add=True)

    def scat_wait(r, dbuf):
        pltpu.make_async_copy(rows[r], acc.at[dbuf.at[0]], ss[r]).wait()

    def step(j, r, dbuf, k, first=False, ahead=True):
        gather_wait(j, r)
        scat(r, dbuf, k)
        if not first:
            scat_wait((r + 2) % 3, dbuf)
        if ahead:
            gather(j + 2, (r + 2) % 3)

    gather(0, 0)
    gather(1, 1)

    for g in range(NGROUPS):
        p = g % 2
        base = g * GROUP
        if g > 0:
            pltpu.make_async_copy(dst_hbm.at[w, g], dv[p], psem).wait()
        step(base, 0, dv[p], 0, first=(g == 0))
        if g + 1 < NGROUPS:
            pltpu.async_copy(dst_hbm.at[w, g + 1], dv[1 - p], psem)
        step(base + 1, 1, dv[p], 1)
        step(base + 2, 2, dv[p], 2)

        if g + 1 < NGROUPS:
            def body(t, carry, _base=base, _dbuf=dv[p]):
                k = 3 * t
                step(_base + k, 0, _dbuf, k)
                step(_base + k + 1, 1, _dbuf, k + 1)
                step(_base + k + 2, 2, _dbuf, k + 2)
                return carry

            lax.fori_loop(1, GROUP // 3, body, 0)
        else:
            def body(t, carry, _base=base, _dbuf=dv[p]):
                k = 3 * t
                step(_base + k, 0, _dbuf, k)
                step(_base + k + 1, 1, _dbuf, k + 1)
                step(_base + k + 2, 2, _dbuf, k + 2)
                return carry

            lax.fori_loop(1, GROUP // 3 - 1, body, 0)
            k = GROUP - 3
            step(base + k, 0, dv[p], k)
            step(base + k + 1, 1, dv[p], k + 1, ahead=False)
            step(base + k + 2, 2, dv[p], k + 2, ahead=False)
            scat_wait(2, dv[p])

    plsc.subcore_barrier()
    pltpu.sync_copy(acc.at[zslice], out_hbm.at[c, zslice])


BLK = 1000


def _tc_body(s0_ref, s1_ref, w1_ref, w2a_ref, w2b_ref, out1_ref, out2_ref):
    sblk = s0_ref[0] + s1_ref[0]
    out1_ref[...] = jnp.dot(sblk, w1_ref[...], preferred_element_type=jnp.float32)
    w2 = w2a_ref[...] + w2b_ref[...]
    out2_ref[...] = jax.nn.sigmoid(
        jnp.dot(sblk, w2, preferred_element_type=jnp.float32))


_tc_finish = pl.pallas_call(
    _tc_body,
    grid=(N_NODES // BLK,),
    in_specs=[
        pl.BlockSpec((1, BLK, D), lambda i: (0, i, 0)),
        pl.BlockSpec((1, BLK, D), lambda i: (1, i, 0)),
        pl.BlockSpec((D, D), lambda i: (0, 0)),
        pl.BlockSpec((D, D), lambda i: (0, 0)),
        pl.BlockSpec((D, D), lambda i: (0, 0)),
    ],
    out_specs=[
        pl.BlockSpec((BLK, D), lambda i: (i, 0)),
        pl.BlockSpec((BLK, D), lambda i: (i, 0)),
    ],
    out_shape=[
        jax.ShapeDtypeStruct((N_NODES, D), jnp.float32),
        jax.ShapeDtypeStruct((N_NODES, D), jnp.float32),
    ],
)


def kernel(x, edge_index, W1, W2a, W2b):
    pad = EPW_PAD - EDGES_PER_WORKER
    src = jnp.pad(edge_index[0].reshape(N_WORKERS, EDGES_PER_WORKER),
                  ((0, 0), (0, pad))).reshape(N_WORKERS * EPW_PAD)
    dst = jnp.pad(edge_index[1].reshape(N_WORKERS, EDGES_PER_WORKER),
                  ((0, 0), (0, pad)), constant_values=N_PAD - 1)
    dst = dst.reshape(N_WORKERS, NGROUPS, GROUP, CHUNK)
    zeros = jnp.zeros((N_PAD, D), jnp.float32)
    partials = _sc_segment_sum(x, src, dst, zeros)
    out1, out2 = _tc_finish(partials, partials, W1, W2a, W2b)
    return (out1, out2)


# speedup vs baseline: 1.6892x; 1.6892x over previous
import functools

import jax
import jax.numpy as jnp
from jax import lax
from jax.experimental import pallas as pl
from jax.experimental.pallas import tpu as pltpu
from jax.experimental.pallas import tpu_sc as plsc

N_NODES = 10000
N_EDGES = 320000
D = 128

NC = 2
NS = 16
N_WORKERS = NC * NS

CHUNK = 80
EDGES_PER_WORKER = N_EDGES // N_WORKERS
GROUP = 25
NGROUPS = 5
CHUNKS_PER_WORKER = GROUP * NGROUPS
ROWS_PER_TILE = 632
N_PAD = NS * ROWS_PER_TILE

_mesh = plsc.VectorSubcoreMesh(core_axis_name="c", subcore_axis_name="s")


@functools.partial(
    pl.kernel,
    mesh=_mesh,
    out_type=jax.ShapeDtypeStruct((NC, N_PAD, D), jnp.float32),
    scratch_types=[
        pltpu.VMEM_SHARED((N_PAD, D), jnp.float32),
        pltpu.VMEM((EDGES_PER_WORKER,), jnp.int32),
        pltpu.VMEM((GROUP, CHUNK), jnp.int32),
        pltpu.VMEM((GROUP, CHUNK), jnp.int32),
        pltpu.VMEM((CHUNK, D), jnp.float32),
        pltpu.VMEM((CHUNK, D), jnp.float32),
        pltpu.VMEM((CHUNK, D), jnp.float32),
        pltpu.SemaphoreType.DMA,
        pltpu.SemaphoreType.DMA,
        pltpu.SemaphoreType.DMA,
        pltpu.SemaphoreType.DMA,
        pltpu.SemaphoreType.DMA,
        pltpu.SemaphoreType.DMA,
        pltpu.SemaphoreType.DMA,
    ],
)
def _sc_segment_sum(x_hbm, src_hbm, dst_hbm, zeros_hbm, out_hbm,
                    acc, src_v, dv0, dv1, r0, r1, r2,
                    g0, g1, g2, s0, s1, s2, psem):
    c = lax.axis_index("c")
    s = lax.axis_index("s")
    w = c * NS + s
    rows = (r0, r1, r2)
    gs = (g0, g1, g2)
    ss = (s0, s1, s2)
    dv = (dv0, dv1)

    zslice = pl.ds(s * ROWS_PER_TILE, ROWS_PER_TILE)
    eslice = pl.ds(w * EDGES_PER_WORKER, EDGES_PER_WORKER)
    pltpu.async_copy(zeros_hbm.at[zslice], acc.at[zslice], psem)
    pltpu.async_copy(src_hbm.at[eslice], src_v, g0)
    pltpu.async_copy(dst_hbm.at[w, 0], dv0, g1)
    pltpu.make_async_copy(zeros_hbm.at[zslice], acc.at[zslice], psem).wait()
    pltpu.make_async_copy(src_hbm.at[eslice], src_v, g0).wait()
    pltpu.make_async_copy(dst_hbm.at[w, 0], dv0, g1).wait()
    plsc.subcore_barrier()

    def _sidx(j):
        return src_v.at[pl.ds(pl.multiple_of(j * CHUNK, 8), CHUNK)]

    def gather(j, r):
        pltpu.async_copy(x_hbm.at[_sidx(j)], rows[r], gs[r])

    def gather_wait(j, r):
        pltpu.make_async_copy(x_hbm.at[_sidx(j)], rows[r], gs[r]).wait()

    def scat(r, dbuf, k):
        pltpu.async_copy(rows[r], acc.at[dbuf.at[k]], ss[r], add=True)

    def scat_wait(r, dbuf):
        pltpu.make_async_copy(rows[r], acc.at[dbuf.at[0]], ss[r]).wait()

    def step(j, r, dbuf, k, first=False, ahead=True):
        if not first:
            scat_wait((r + 2) % 3, dbuf)
        if ahead:
            gather(j + 2, (r + 2) % 3)
        gather_wait(j, r)
        scat(r, dbuf, k)

    gather(0, 0)
    gather(1, 1)

    for g in range(NGROUPS):
        p = g % 2
        base = g * GROUP
        if g > 0:
            pltpu.make_async_copy(dst_hbm.at[w, g], dv[p], psem).wait()
        step(base, base % 3, dv[p], 0, first=(g == 0))
        if g + 1 < NGROUPS:
            pltpu.async_copy(dst_hbm.at[w, g + 1], dv[1 - p], psem)

            def body(t, carry, _base=base, _dbuf=dv[p]):
                k = 3 * t + 1
                step(_base + k, (_base + 1) % 3, _dbuf, k)
                step(_base + k + 1, (_base + 2) % 3, _dbuf, k + 1)
                step(_base + k + 2, _base % 3, _dbuf, k + 2)
                return carry

            lax.fori_loop(0, (GROUP - 1) // 3, body, 0)
        else:
            def body(t, carry, _base=base, _dbuf=dv[p]):
                k = 3 * t + 1
                step(_base + k, (_base + 1) % 3, _dbuf, k)
                step(_base + k + 1, (_base + 2) % 3, _dbuf, k + 1)
                step(_base + k + 2, _base % 3, _dbuf, k + 2)
                return carry

            lax.fori_loop(0, (GROUP - 4) // 3, body, 0)
            k = GROUP - 3
            step(base + k, (base + k) % 3, dv[p], k)
            step(base + k + 1, (base + k + 1) % 3, dv[p], k + 1, ahead=False)
            step(base + k + 2, (base + k + 2) % 3, dv[p], k + 2, ahead=False)
            scat_wait((base + k + 2) % 3, dv[p])

    plsc.subcore_barrier()
    pltpu.sync_copy(acc.at[zslice], out_hbm.at[c, zslice])


BLK = 1000


def _tc_body(s0_ref, s1_ref, w1_ref, w2a_ref, w2b_ref, out1_ref, out2_ref):
    sblk = s0_ref[0] + s1_ref[0]
    out1_ref[...] = jnp.dot(sblk, w1_ref[...], preferred_element_type=jnp.float32)
    w2 = w2a_ref[...] + w2b_ref[...]
    out2_ref[...] = jax.nn.sigmoid(
        jnp.dot(sblk, w2, preferred_element_type=jnp.float32))


_tc_finish = pl.pallas_call(
    _tc_body,
    grid=(N_NODES // BLK,),
    in_specs=[
        pl.BlockSpec((1, BLK, D), lambda i: (0, i, 0)),
        pl.BlockSpec((1, BLK, D), lambda i: (1, i, 0)),
        pl.BlockSpec((D, D), lambda i: (0, 0)),
        pl.BlockSpec((D, D), lambda i: (0, 0)),
        pl.BlockSpec((D, D), lambda i: (0, 0)),
    ],
    out_specs=[
        pl.BlockSpec((BLK, D), lambda i: (i, 0)),
        pl.BlockSpec((BLK, D), lambda i: (i, 0)),
    ],
    out_shape=[
        jax.ShapeDtypeStruct((N_NODES, D), jnp.float32),
        jax.ShapeDtypeStruct((N_NODES, D), jnp.float32),
    ],
)


def kernel(x, edge_index, W1, W2a, W2b):
    src = edge_index[0]
    dst = edge_index[1].reshape(N_WORKERS, NGROUPS, GROUP, CHUNK)
    zeros = jnp.zeros((N_PAD, D), jnp.float32)
    partials = _sc_segment_sum(x, src, dst, zeros)
    out1, out2 = _tc_finish(partials, partials, W1, W2a, W2b)
    return (out1, out2)


# speedup vs baseline: 1.6897x; 1.0003x over previous
import functools

import jax
import jax.numpy as jnp
from jax import lax
from jax.experimental import pallas as pl
from jax.experimental.pallas import tpu as pltpu
from jax.experimental.pallas import tpu_sc as plsc

N_NODES = 10000
N_EDGES = 320000
D = 128

NC = 2
NS = 16
N_WORKERS = NC * NS

CHUNK = 80
EDGES_PER_WORKER = N_EDGES // N_WORKERS
GROUP = 25
NGROUPS = 5
CHUNKS_PER_WORKER = GROUP * NGROUPS
ROWS_PER_TILE = 632
N_PAD = NS * ROWS_PER_TILE

_mesh = plsc.VectorSubcoreMesh(core_axis_name="c", subcore_axis_name="s")


@functools.partial(
    pl.kernel,
    mesh=_mesh,
    out_type=jax.ShapeDtypeStruct((NC, N_PAD, D), jnp.float32),
    scratch_types=[
        pltpu.VMEM_SHARED((N_PAD, D), jnp.float32),
        pltpu.VMEM((EDGES_PER_WORKER,), jnp.int32),
        pltpu.VMEM((GROUP, CHUNK), jnp.int32),
        pltpu.VMEM((GROUP, CHUNK), jnp.int32),
        pltpu.VMEM((CHUNK, D), jnp.float32),
        pltpu.VMEM((CHUNK, D), jnp.float32),
        pltpu.VMEM((CHUNK, D), jnp.float32),
        pltpu.SemaphoreType.DMA,
        pltpu.SemaphoreType.DMA,
        pltpu.SemaphoreType.DMA,
        pltpu.SemaphoreType.DMA,
        pltpu.SemaphoreType.DMA,
        pltpu.SemaphoreType.DMA,
        pltpu.SemaphoreType.DMA,
    ],
)
def _sc_segment_sum(x_hbm, src_hbm, dst_hbm, zeros_hbm, out_hbm,
                    acc, src_v, dv0, dv1, r0, r1, r2,
                    g0, g1, g2, s0, s1, s2, psem):
    c = lax.axis_index("c")
    s = lax.axis_index("s")
    w = c * NS + s
    rows = (r0, r1, r2)
    gs = (g0, g1, g2)
    ss = (s0, s1, s2)
    dv = (dv0, dv1)

    zslice = pl.ds(s * ROWS_PER_TILE, ROWS_PER_TILE)
    eslice = pl.ds(w * EDGES_PER_WORKER, EDGES_PER_WORKER)
    pltpu.async_copy(zeros_hbm.at[zslice], acc.at[zslice], psem)
    pltpu.async_copy(src_hbm.at[eslice], src_v, g0)
    pltpu.async_copy(dst_hbm.at[w, 0], dv0, g1)
    pltpu.make_async_copy(zeros_hbm.at[zslice], acc.at[zslice], psem).wait()
    pltpu.make_async_copy(src_hbm.at[eslice], src_v, g0).wait()
    pltpu.make_async_copy(dst_hbm.at[w, 0], dv0, g1).wait()
    plsc.subcore_barrier()

    H = CHUNK // 2

    def _sidx(j, h):
        return src_v.at[pl.ds(pl.multiple_of(j * CHUNK + h * H, 8), H)]

    def gather(j, r):
        pltpu.async_copy(x_hbm.at[_sidx(j, 0)], rows[r].at[pl.ds(0, H)], gs[r])
        pltpu.async_copy(x_hbm.at[_sidx(j, 1)], rows[r].at[pl.ds(H, H)], gs[r])

    def gather_wait(j, r):
        pltpu.make_async_copy(x_hbm.at[_sidx(j, 0)], rows[r].at[pl.ds(0, H)], gs[r]).wait()
        pltpu.make_async_copy(x_hbm.at[_sidx(j, 1)], rows[r].at[pl.ds(H, H)], gs[r]).wait()

    def scat(r, dbuf, k):
        pltpu.async_copy(rows[r], acc.at[dbuf.at[k]], ss[r], add=True)

    def scat_wait(r, dbuf):
        pltpu.make_async_copy(rows[r], acc.at[dbuf.at[0]], ss[r]).wait()

    def step(j, r, dbuf, k, first=False, ahead=True):
        if not first:
            scat_wait((r + 2) % 3, dbuf)
        if ahead:
            gather(j + 2, (r + 2) % 3)
        gather_wait(j, r)
        scat(r, dbuf, k)

    gather(0, 0)
    gather(1, 1)

    for g in range(NGROUPS):
        p = g % 2
        base = g * GROUP
        if g > 0:
            pltpu.make_async_copy(dst_hbm.at[w, g], dv[p], psem).wait()
        step(base, base % 3, dv[p], 0, first=(g == 0))
        if g + 1 < NGROUPS:
            pltpu.async_copy(dst_hbm.at[w, g + 1], dv[1 - p], psem)

            def body(t, carry, _base=base, _dbuf=dv[p]):
                k = 3 * t + 1
                step(_base + k, (_base + 1) % 3, _dbuf, k)
                step(_base + k + 1, (_base + 2) % 3, _dbuf, k + 1)
                step(_base + k + 2, _base % 3, _dbuf, k + 2)
                return carry

            lax.fori_loop(0, (GROUP - 1) // 3, body, 0)
        else:
            def body(t, carry, _base=base, _dbuf=dv[p]):
                k = 3 * t + 1
                step(_base + k, (_base + 1) % 3, _dbuf, k)
                step(_base + k + 1, (_base + 2) % 3, _dbuf, k + 1)
                step(_base + k + 2, _base % 3, _dbuf, k + 2)
                return carry

            lax.fori_loop(0, (GROUP - 4) // 3, body, 0)
            k = GROUP - 3
            step(base + k, (base + k) % 3, dv[p], k)
            step(base + k + 1, (base + k + 1) % 3, dv[p], k + 1, ahead=False)
            step(base + k + 2, (base + k + 2) % 3, dv[p], k + 2, ahead=False)
            scat_wait((base + k + 2) % 3, dv[p])

    plsc.subcore_barrier()
    pltpu.sync_copy(acc.at[zslice], out_hbm.at[c, zslice])


BLK = 1000


def _tc_body(s0_ref, s1_ref, w1_ref, w2a_ref, w2b_ref, out1_ref, out2_ref):
    sblk = s0_ref[0] + s1_ref[0]
    out1_ref[...] = jnp.dot(sblk, w1_ref[...], preferred_element_type=jnp.float32)
    w2 = w2a_ref[...] + w2b_ref[...]
    out2_ref[...] = jax.nn.sigmoid(
        jnp.dot(sblk, w2, preferred_element_type=jnp.float32))


_tc_finish = pl.pallas_call(
    _tc_body,
    grid=(N_NODES // BLK,),
    in_specs=[
        pl.BlockSpec((1, BLK, D), lambda i: (0, i, 0)),
        pl.BlockSpec((1, BLK, D), lambda i: (1, i, 0)),
        pl.BlockSpec((D, D), lambda i: (0, 0)),
        pl.BlockSpec((D, D), lambda i: (0, 0)),
        pl.BlockSpec((D, D), lambda i: (0, 0)),
    ],
    out_specs=[
        pl.BlockSpec((BLK, D), lambda i: (i, 0)),
        pl.BlockSpec((BLK, D), lambda i: (i, 0)),
    ],
    out_shape=[
        jax.ShapeDtypeStruct((N_NODES, D), jnp.float32),
        jax.ShapeDtypeStruct((N_NODES, D), jnp.float32),
    ],
)


def kernel(x, edge_index, W1, W2a, W2b):
    src = edge_index[0]
    dst = edge_index[1].reshape(N_WORKERS, NGROUPS, GROUP, CHUNK)
    zeros = jnp.zeros((N_PAD, D), jnp.float32)
    partials = _sc_segment_sum(x, src, dst, zeros)
    out1, out2 = _tc_finish(partials, partials, W1, W2a, W2b)
    return (out1, out2)


# speedup vs baseline: 1.7061x; 1.0097x over previous
import functools

import jax
import jax.numpy as jnp
from jax import lax
from jax.experimental import pallas as pl
from jax.experimental.pallas import tpu as pltpu
from jax.experimental.pallas import tpu_sc as plsc

N_NODES = 10000
N_EDGES = 320000
D = 128

NC = 2
NS = 16
N_WORKERS = NC * NS

CHUNK = 80
EDGES_PER_WORKER = N_EDGES // N_WORKERS
GROUP = 25
NGROUPS = 5
CHUNKS_PER_WORKER = GROUP * NGROUPS
ROWS_PER_TILE = 632
N_PAD = NS * ROWS_PER_TILE

_mesh = plsc.VectorSubcoreMesh(core_axis_name="c", subcore_axis_name="s")


@functools.partial(
    pl.kernel,
    mesh=_mesh,
    out_type=jax.ShapeDtypeStruct((NC, N_PAD, D), jnp.float32),
    scratch_types=[
        pltpu.VMEM_SHARED((N_PAD, D), jnp.float32),
        pltpu.VMEM((EDGES_PER_WORKER,), jnp.int32),
        pltpu.VMEM((GROUP, CHUNK), jnp.int32),
        pltpu.VMEM((GROUP, CHUNK), jnp.int32),
        pltpu.VMEM((CHUNK, D), jnp.float32),
        pltpu.VMEM((CHUNK, D), jnp.float32),
        pltpu.VMEM((CHUNK, D), jnp.float32),
        pltpu.SemaphoreType.DMA,
        pltpu.SemaphoreType.DMA,
        pltpu.SemaphoreType.DMA,
        pltpu.SemaphoreType.DMA,
        pltpu.SemaphoreType.DMA,
        pltpu.SemaphoreType.DMA,
        pltpu.SemaphoreType.DMA,
    ],
)
def _sc_segment_sum(x_hbm, src_hbm, dst_hbm, zeros_hbm, out_hbm,
                    acc, src_v, dv0, dv1, r0, r1, r2,
                    g0, g1, g2, s0, s1, s2, psem):
    c = lax.axis_index("c")
    s = lax.axis_index("s")
    w = c * NS + s
    rows = (r0, r1, r2)
    gs = (g0, g1, g2)
    ss = (s0, s1, s2)
    dv = (dv0, dv1)

    zslice = pl.ds(s * ROWS_PER_TILE, ROWS_PER_TILE)
    eslice = pl.ds(w * EDGES_PER_WORKER, EDGES_PER_WORKER)
    pltpu.async_copy(zeros_hbm.at[zslice], acc.at[zslice], psem)
    pltpu.async_copy(src_hbm.at[eslice], src_v, s0)
    pltpu.async_copy(dst_hbm.at[w, 0], dv0, s1)
    pltpu.make_async_copy(src_hbm.at[eslice], src_v, s0).wait()

    H = CHUNK // 2

    def _sidx(j, h):
        return src_v.at[pl.ds(pl.multiple_of(j * CHUNK + h * H, 8), H)]

    def gather(j, r):
        pltpu.async_copy(x_hbm.at[_sidx(j, 0)], rows[r].at[pl.ds(0, H)], gs[r])
        pltpu.async_copy(x_hbm.at[_sidx(j, 1)], rows[r].at[pl.ds(H, H)], gs[r])

    def gather_wait(j, r):
        pltpu.make_async_copy(x_hbm.at[_sidx(j, 0)], rows[r].at[pl.ds(0, H)], gs[r]).wait()
        pltpu.make_async_copy(x_hbm.at[_sidx(j, 1)], rows[r].at[pl.ds(H, H)], gs[r]).wait()

    def scat(r, dbuf, k):
        pltpu.async_copy(rows[r], acc.at[dbuf.at[k]], ss[r], add=True)

    def scat_wait(r, dbuf):
        pltpu.make_async_copy(rows[r], acc.at[dbuf.at[0]], ss[r]).wait()

    def step(j, r, dbuf, k, first=False, ahead=True):
        if not first:
            scat_wait((r + 2) % 3, dbuf)
        if ahead:
            gather(j + 2, (r + 2) % 3)
        gather_wait(j, r)
        scat(r, dbuf, k)

    gather(0, 0)
    gather(1, 1)
    pltpu.make_async_copy(zeros_hbm.at[zslice], acc.at[zslice], psem).wait()
    pltpu.make_async_copy(dst_hbm.at[w, 0], dv0, s1).wait()
    plsc.subcore_barrier()

    for g in range(NGROUPS):
        p = g % 2
        base = g * GROUP
        if g > 0:
            pltpu.make_async_copy(dst_hbm.at[w, g], dv[p], psem).wait()
        step(base, base % 3, dv[p], 0, first=(g == 0))
        if g + 1 < NGROUPS:
            pltpu.async_copy(dst_hbm.at[w, g + 1], dv[1 - p], psem)

            def body(t, carry, _base=base, _dbuf=dv[p]):
                k = 3 * t + 1
                step(_base + k, (_base + 1) % 3, _dbuf, k)
                step(_base + k + 1, (_base + 2) % 3, _dbuf, k + 1)
                step(_base + k + 2, _base % 3, _dbuf, k + 2)
                return carry

            lax.fori_loop(0, (GROUP - 1) // 3, body, 0)
        else:
            def body(t, carry, _base=base, _dbuf=dv[p]):
                k = 3 * t + 1
                step(_base + k, (_base + 1) % 3, _dbuf, k)
                step(_base + k + 1, (_base + 2) % 3, _dbuf, k + 1)
                step(_base + k + 2, _base % 3, _dbuf, k + 2)
                return carry

            lax.fori_loop(0, (GROUP - 4) // 3, body, 0)
            k = GROUP - 3
            step(base + k, (base + k) % 3, dv[p], k)
            step(base + k + 1, (base + k + 1) % 3, dv[p], k + 1, ahead=False)
            step(base + k + 2, (base + k + 2) % 3, dv[p], k + 2, ahead=False)
            scat_wait((base + k + 2) % 3, dv[p])

    plsc.subcore_barrier()
    pltpu.sync_copy(acc.at[zslice], out_hbm.at[c, zslice])


BLK = 1000


def _tc_body(s0_ref, s1_ref, w1_ref, w2a_ref, w2b_ref, out1_ref, out2_ref):
    sblk = s0_ref[0] + s1_ref[0]
    out1_ref[...] = jnp.dot(sblk, w1_ref[...], preferred_element_type=jnp.float32)
    w2 = w2a_ref[...] + w2b_ref[...]
    out2_ref[...] = jax.nn.sigmoid(
        jnp.dot(sblk, w2, preferred_element_type=jnp.float32))


_tc_finish = pl.pallas_call(
    _tc_body,
    grid=(N_NODES // BLK,),
    in_specs=[
        pl.BlockSpec((1, BLK, D), lambda i: (0, i, 0)),
        pl.BlockSpec((1, BLK, D), lambda i: (1, i, 0)),
        pl.BlockSpec((D, D), lambda i: (0, 0)),
        pl.BlockSpec((D, D), lambda i: (0, 0)),
        pl.BlockSpec((D, D), lambda i: (0, 0)),
    ],
    out_specs=[
        pl.BlockSpec((BLK, D), lambda i: (i, 0)),
        pl.BlockSpec((BLK, D), lambda i: (i, 0)),
    ],
    out_shape=[
        jax.ShapeDtypeStruct((N_NODES, D), jnp.float32),
        jax.ShapeDtypeStruct((N_NODES, D), jnp.float32),
    ],
)


def kernel(x, edge_index, W1, W2a, W2b):
    src = edge_index[0]
    dst = edge_index[1].reshape(N_WORKERS, NGROUPS, GROUP, CHUNK)
    zeros = jnp.zeros((N_PAD, D), jnp.float32)
    partials = _sc_segment_sum(x, src, dst, zeros)
    out1, out2 = _tc_finish(partials, partials, W1, W2a, W2b)
    return (out1, out2)
